# baseline (speedup 1.0000x reference)
import functools

import jax
import jax.numpy as jnp
from jax import lax
from jax.experimental import pallas as pl
from jax.experimental.pallas import tpu as pltpu
from jax.experimental.pallas import tpu_sc as plsc

BATCH = 16384
EMBED_DIM = 64
NUM_CORES = 2
NUM_SUBCORES = 16
NUM_WORKERS = NUM_CORES * NUM_SUBCORES
BPW = BATCH // NUM_WORKERS
CHUNK = 128
NCHUNKS = BPW // CHUNK
LANES = 16
SUBS = EMBED_DIM // LANES


def _tec_body(h_hbm, r_hbm, t_hbm, ent_hbm, rel_hbm, out_hbm,
              hidx, ridx, tidx, hrows, rrows, trows, sem):
    wid = lax.axis_index("s") * NUM_CORES + lax.axis_index("c")
    base = wid * BPW

    pltpu.sync_copy(h_hbm.at[pl.ds(base, BPW)], hidx)
    pltpu.sync_copy(r_hbm.at[pl.ds(base, BPW)], ridx)
    pltpu.sync_copy(t_hbm.at[pl.ds(base, BPW)], tidx)

    copies = []
    for j in range(NCHUNKS):
        sl = pl.ds(j * CHUNK, CHUNK)
        copies.append(pltpu.async_copy(ent_hbm.at[hidx.at[sl]], hrows.at[sl], sem))
        copies.append(pltpu.async_copy(rel_hbm.at[ridx.at[sl]], rrows.at[sl], sem))
        copies.append(pltpu.async_copy(ent_hbm.at[tidx.at[sl]], trows.at[sl], sem))
    for c in copies:
        c.wait()

    def row_body(i, carry):
        for c in range(SUBS):
            s = pl.ds(c * LANES, LANES)
            hrows[i, s] = hrows[i, s] + rrows[i, s] - trows[i, s]
        return carry

    lax.fori_loop(0, BPW, row_body, 0)

    pltpu.sync_copy(hrows, out_hbm.at[pl.ds(base, BPW)])


def kernel(h, r, t, entity_emb, relation_emb):
    mesh = plsc.VectorSubcoreMesh(core_axis_name="c", subcore_axis_name="s")
    k = functools.partial(
        pl.kernel,
        mesh=mesh,
        compiler_params=pltpu.CompilerParams(use_tc_tiling_on_sc=False),
        out_type=jax.ShapeDtypeStruct((BATCH, EMBED_DIM), jnp.float32),
        scratch_types=[
            pltpu.VMEM((BPW,), jnp.int32),
            pltpu.VMEM((BPW,), jnp.int32),
            pltpu.VMEM((BPW,), jnp.int32),
            pltpu.VMEM((BPW, EMBED_DIM), jnp.float32),
            pltpu.VMEM((BPW, EMBED_DIM), jnp.float32),
            pltpu.VMEM((BPW, EMBED_DIM), jnp.float32),
            pltpu.SemaphoreType.DMA,
        ],
    )(_tec_body)
    return k(h, r, t, entity_emb, relation_emb)
